# baseline (speedup 1.0000x reference)
import functools

import jax
import jax.numpy as jnp
from jax import lax
from jax.experimental import pallas as pl
from jax.experimental.pallas import tpu as pltpu
from jax.experimental.pallas import tpu_sc as plsc

N = 10000
E = 320000
F = 128
H = 64

NC = 2
NS = 16
NW = NC * NS
B = 32
NB = 324
EW = NB * B
EPAD = NW * EW
BR = 128
NBR = EW // BR
NACC = ((N + 1 + NS * 8 - 1) // (NS * 8)) * NS * 8
NPAD_ROWS = NACC - N
ROWS_PER_SUB = NACC // NS

EB = 4096


import numpy as _np

_PERM = _np.empty((F,), dtype=_np.int32)
for _q in range(F // 32):
    for _i in range(16):
        _PERM[32 * _q + _i] = 32 * _q + 2 * _i
        _PERM[32 * _q + 16 + _i] = 32 * _q + 2 * _i + 1
_UNPERM_MAT = _np.eye(F, dtype=_np.float32)[_PERM]


@functools.lru_cache(maxsize=None)
def _sc_mesh():
    return plsc.VectorSubcoreMesh(core_axis_name="c", subcore_axis_name="s")


def _rel_body(cx_hbm, cy_hbm, cz_hbm, src_hbm, dst_hbm, rx_hbm, ry_hbm, rz_hbm,
              srcs_v, dsts_v, s0_v, s1_v, s2_v, d0_v, d1_v, d2_v,
              o0_v, o1_v, o2_v, sem, semw):
    c = lax.axis_index("c")
    s = lax.axis_index("s")
    wid = s * NC + c
    pltpu.sync_copy(src_hbm.at[wid], srcs_v)
    pltpu.sync_copy(dst_hbm.at[wid], dsts_v)

    halves = ((s0_v.at[pl.ds(0, BR)], s1_v.at[pl.ds(0, BR)],
               s2_v.at[pl.ds(0, BR)], d0_v.at[pl.ds(0, BR)],
               d1_v.at[pl.ds(0, BR)], d2_v.at[pl.ds(0, BR)]),
              (s0_v.at[pl.ds(BR, BR)], s1_v.at[pl.ds(BR, BR)],
               s2_v.at[pl.ds(BR, BR)], d0_v.at[pl.ds(BR, BR)],
               d1_v.at[pl.ds(BR, BR)], d2_v.at[pl.ds(BR, BR)]))

    def copies(t, p):
        si = srcs_v.at[pl.ds(t * BR, BR)]
        di = dsts_v.at[pl.ds(t * BR, BR)]
        b0, b1, b2, b3, b4, b5 = halves[p]
        return (
            pltpu.make_async_copy(cx_hbm.at[si], b0, sem),
            pltpu.make_async_copy(cy_hbm.at[si], b1, sem),
            pltpu.make_async_copy(cz_hbm.at[si], b2, sem),
            pltpu.make_async_copy(cx_hbm.at[di], b3, sem),
            pltpu.make_async_copy(cy_hbm.at[di], b4, sem),
            pltpu.make_async_copy(cz_hbm.at[di], b5, sem),
        )

    out_halves = ((o0_v.at[pl.ds(0, BR)], o1_v.at[pl.ds(0, BR)],
                   o2_v.at[pl.ds(0, BR)]),
                  (o0_v.at[pl.ds(BR, BR)], o1_v.at[pl.ds(BR, BR)],
                   o2_v.at[pl.ds(BR, BR)]))

    def writes(t, p):
        base = wid * EW + t * BR
        oh = out_halves[p]
        return (
            pltpu.make_async_copy(oh[0], rx_hbm.at[pl.ds(base, BR)], semw),
            pltpu.make_async_copy(oh[1], ry_hbm.at[pl.ds(base, BR)], semw),
            pltpu.make_async_copy(oh[2], rz_hbm.at[pl.ds(base, BR)], semw),
        )

    for cp in copies(0, 0):
        cp.start()
    for cp in copies(1, 1):
        cp.start()

    def step(j, p, first):
        for cp in copies(j, p):
            cp.wait()
        if not first:
            for cp in writes(j - 2, p):
                cp.wait()
        off = p * BR
        for sbuf, dbuf, obuf in ((s0_v, d0_v, o0_v), (s1_v, d1_v, o1_v),
                                 (s2_v, d2_v, o2_v)):
            for k in range(BR // 16):
                sl = pl.ds(off + k * 16, 16)
                obuf[sl] = sbuf[sl] - dbuf[sl]
        for cp in writes(j, p):
            cp.start()

        @pl.when(j + 2 < NBR)
        def _():
            for cp in copies(j + 2, p):
                cp.start()

    step(0, 0, True)
    step(1, 1, True)

    def pair(kk, carry):
        for p in (0, 1):
            j = 2 * kk + p
            step(j, p, False)
        return carry

    lax.fori_loop(1, (NBR - 1) // 2, pair, 0)
    step(NBR - 1, (NBR - 1) % 2, False)
    for cp in writes(NBR - 2, (NBR - 2) % 2):
        cp.wait()
    for cp in writes(NBR - 1, (NBR - 1) % 2):
        cp.wait()


@functools.lru_cache(maxsize=None)
def _rel_call():
    return pl.kernel(
        _rel_body,
        out_type=tuple(jax.ShapeDtypeStruct((EPAD,), jnp.float32)
                       for _ in range(3)),
        mesh=_sc_mesh(),
        scratch_types=[
            pltpu.VMEM((EW,), jnp.int32),
            pltpu.VMEM((EW,), jnp.int32),
            pltpu.VMEM((2 * BR,), jnp.float32),
            pltpu.VMEM((2 * BR,), jnp.float32),
            pltpu.VMEM((2 * BR,), jnp.float32),
            pltpu.VMEM((2 * BR,), jnp.float32),
            pltpu.VMEM((2 * BR,), jnp.float32),
            pltpu.VMEM((2 * BR,), jnp.float32),
            pltpu.VMEM((2 * BR,), jnp.float32),
            pltpu.VMEM((2 * BR,), jnp.float32),
            pltpu.VMEM((2 * BR,), jnp.float32),
            pltpu.SemaphoreType.DMA,
            pltpu.SemaphoreType.DMA,
        ],
    )


def _msg_body(w_hbm, m_hbm, src_hbm, dst_hbm, zero_hbm, out_hbm,
              srcs_v, d0_v, d1_v, d2_v, d3_v, w0_v, w1_v, w2_v, w3_v,
              m0_v, m1_v, m2_v, m3_v, acc_sh,
              sd0, sd1, sd2, sd3, sg0, sg1, sg2, sg3,
              sw0, sw1, sw2, sw3, ss0, ss1, ss2, ss3):
    c = lax.axis_index("c")
    s = lax.axis_index("s")
    wid = s * NC + c
    pltpu.sync_copy(src_hbm.at[wid], srcs_v)
    pltpu.sync_copy(zero_hbm.at[pl.ds(s * ROWS_PER_SUB, ROWS_PER_SUB)],
                    acc_sh.at[pl.ds(s * ROWS_PER_SUB, ROWS_PER_SUB)])

    bufs = ((d0_v, w0_v, m0_v, sd0, sg0, sw0, ss0),
            (d1_v, w1_v, m1_v, sd1, sg1, sw1, ss1),
            (d2_v, w2_v, m2_v, sd2, sg2, sw2, ss2),
            (d3_v, w3_v, m3_v, sd3, sg3, sw3, ss3))

    def copies(t, p):
        d_v, w_v, m_v, sd, sg, sw, _ = bufs[p]
        return (
            pltpu.make_async_copy(dst_hbm.at[wid, pl.ds(t * B, B)], d_v, sd),
            pltpu.make_async_copy(m_hbm.at[srcs_v.at[pl.ds(t * B, B)]], m_v, sg),
            pltpu.make_async_copy(w_hbm.at[pl.ds(wid * EW + t * B, B)], w_v, sw),
        )

    def scatter_cp(p):
        d_v, _, m_v, _, _, _, ss = bufs[p]
        return pltpu.make_async_copy(m_v, acc_sh.at[d_v], ss)

    for t in (0, 1, 2):
        for cp in copies(t, t):
            cp.start()
    plsc.subcore_barrier()

    def step(j, p, first):
        for cp in copies(j, p):
            cp.wait()
        d_v, w_v, m_v = bufs[p][0], bufs[p][1], bufs[p][2]

        def row(r, c2):
            for g in range(4):
                lo = pl.ds(32 * g, 16)
                hi = pl.ds(32 * g + 16, 16)
                bits = w_v[r, pl.ds(16 * g, 16)]
                flo = lax.bitcast_convert_type(bits << 16, jnp.float32)
                fhi = lax.bitcast_convert_type(bits & jnp.int32(-65536),
                                               jnp.float32)
                m_v[r, lo] = flo * m_v[r, lo]
                m_v[r, hi] = fhi * m_v[r, hi]
            return c2

        lax.fori_loop(0, B, row, 0)
        scatter_cp(p).start(add=True)

        @pl.when(j + 3 < NB)
        def _():
            if first:
                for cp in copies(j + 3, (p + 3) % 4):
                    cp.start()
            else:
                scatter_cp((p + 3) % 4).wait()
                for cp in copies(j + 3, (p + 3) % 4):
                    cp.start()

    step(0, 0, True)

    def quad(k, carry):
        for p in (0, 1, 2, 3):
            j = 4 * k + p

            @pl.when(jnp.logical_and(j >= 1, j < NB))
            def _():
                step(j, p, False)
        return carry

    lax.fori_loop(0, NB // 4 + 1, quad, 0)
    for t in (NB - 4, NB - 3, NB - 2, NB - 1):
        scatter_cp(t % 4).wait()
    plsc.subcore_barrier()
    pltpu.sync_copy(acc_sh.at[pl.ds(s * ROWS_PER_SUB, ROWS_PER_SUB)],
                    out_hbm.at[c, pl.ds(s * ROWS_PER_SUB, ROWS_PER_SUB)])


@functools.lru_cache(maxsize=None)
def _msg_call():
    return pl.kernel(
        _msg_body,
        out_type=jax.ShapeDtypeStruct((NC, NACC, F), jnp.float32),
        mesh=_sc_mesh(),
        scratch_types=[
            pltpu.VMEM((EW,), jnp.int32),
            pltpu.VMEM((B,), jnp.int32),
            pltpu.VMEM((B,), jnp.int32),
            pltpu.VMEM((B,), jnp.int32),
            pltpu.VMEM((B,), jnp.int32),
            pltpu.VMEM((B, F // 2), jnp.int32),
            pltpu.VMEM((B, F // 2), jnp.int32),
            pltpu.VMEM((B, F // 2), jnp.int32),
            pltpu.VMEM((B, F // 2), jnp.int32),
            pltpu.VMEM((B, F), jnp.float32),
            pltpu.VMEM((B, F), jnp.float32),
            pltpu.VMEM((B, F), jnp.float32),
            pltpu.VMEM((B, F), jnp.float32),
            pltpu.VMEM_SHARED((NACC, F), jnp.float32),
        ] + [pltpu.SemaphoreType.DMA] * 16,
    )


def _m0_body(x_ref, w_ref, b_ref, o_ref):
    o_ref[...] = jnp.dot(x_ref[...].astype(jnp.bfloat16), w_ref[...],
                         preferred_element_type=jnp.float32) + b_ref[...]


def _m1_body(p_ref, w_ref, b_ref, o_ref):
    h = (p_ref[0] + p_ref[1])[:N]
    h = jnp.where(h > 0, h, 0.01 * h)
    o_ref[...] = jnp.dot(h.astype(jnp.bfloat16), w_ref[...],
                         preferred_element_type=jnp.float32) + b_ref[...]


def _fin_body(p_ref, M_ref, o_ref):
    o_ref[...] = jnp.dot((p_ref[0] + p_ref[1])[:N], M_ref[...],
                         preferred_element_type=jnp.float32)


def _wmlp_body(rel_ref, W1_ref, b1_ref, W2_ref, b2_ref, w_ref):
    u = lax.dot_general(rel_ref[...], W1_ref[...],
                        dimension_numbers=(((0,), (0,)), ((), ())),
                        preferred_element_type=jnp.float32) + b1_ref[...]
    u = jnp.maximum(u, 0.0)
    w = jnp.dot(u.astype(jnp.bfloat16), W2_ref[...],
                preferred_element_type=jnp.float32) + b2_ref[...]
    w_ref[...] = w.astype(jnp.bfloat16)


def _node_linear(x, Wn, bn):
    return pl.pallas_call(
        _m0_body,
        out_shape=jax.ShapeDtypeStruct((N, F), jnp.float32),
    )(x, Wn, bn.reshape(1, F))


def _node_linear_from_partials(p, Wn, bn):
    return pl.pallas_call(
        _m1_body,
        out_shape=jax.ShapeDtypeStruct((N, F), jnp.float32),
    )(p, Wn, bn.reshape(1, F))


def _combine(p, M):
    return pl.pallas_call(
        _fin_body,
        out_shape=jax.ShapeDtypeStruct((N, F), jnp.float32),
    )(p, M)


def _edge_mlp(rel, We1, be1, We2, be2):
    return pl.pallas_call(
        _wmlp_body,
        grid=(EPAD // EB,),
        in_specs=[
            pl.BlockSpec((3, EB), lambda i: (0, i)),
            pl.BlockSpec((3, H), lambda i: (0, 0)),
            pl.BlockSpec((1, H), lambda i: (0, 0)),
            pl.BlockSpec((H, F), lambda i: (0, 0)),
            pl.BlockSpec((1, F), lambda i: (0, 0)),
        ],
        out_specs=pl.BlockSpec((EB, F), lambda i: (i, 0)),
        out_shape=jax.ShapeDtypeStruct((EPAD, F), jnp.bfloat16),
    )(rel, We1, be1.reshape(1, H), We2, be2.reshape(1, F))


def kernel(x, coords, edge_index, We1_0, be1_0, We2_0, be2_0, Wn_0, bn_0,
           We1_1, be1_1, We2_1, be2_1, Wn_1, bn_1):
    src = edge_index[0]
    dst = edge_index[1]
    pad = EPAD - E
    src_p = jnp.concatenate([src, jnp.arange(pad, dtype=jnp.int32) % 997])
    dummy = N + (jnp.arange(pad, dtype=jnp.int32) % NPAD_ROWS)
    dst_p = jnp.concatenate([dst, dummy])
    src2d = src_p.reshape(NW, EW)
    dst2d = dst_p.reshape(NW, EW)

    cx = coords[:, 0]
    cy = coords[:, 1]
    cz = coords[:, 2]
    zeros_acc = jnp.zeros((NACC, F), jnp.float32)

    rx, ry, rz = _rel_call()(cx, cy, cz, src2d, dst2d)
    rel = jnp.stack([rx, ry, rz])

    perm = jnp.asarray(_PERM)
    Wn0p = Wn_0[:, perm].astype(jnp.bfloat16)
    Wn1p = Wn_1[perm][:, perm].astype(jnp.bfloat16)

    m0 = _node_linear(x, Wn0p, bn_0[perm])
    w0 = _edge_mlp(rel, We1_0, be1_0, We2_0.astype(jnp.bfloat16), be2_0)
    w0i = lax.bitcast_convert_type(w0.reshape(EPAD, F // 2, 2), jnp.int32)
    p0 = _msg_call()(w0i, m0, src2d, dst2d, zeros_acc)

    m1 = _node_linear_from_partials(p0, Wn1p, bn_1[perm])
    w1 = _edge_mlp(rel, We1_1, be1_1, We2_1.astype(jnp.bfloat16), be2_1)
    w1i = lax.bitcast_convert_type(w1.reshape(EPAD, F // 2, 2), jnp.int32)
    p1 = _msg_call()(w1i, m1, src2d, dst2d, zeros_acc)

    return _combine(p1, jnp.asarray(_UNPERM_MAT))


# speedup vs baseline: 3.4116x; 3.4116x over previous
import functools

import jax
import jax.numpy as jnp
from jax import lax
from jax.experimental import pallas as pl
from jax.experimental.pallas import tpu as pltpu
from jax.experimental.pallas import tpu_sc as plsc

N = 10000
E = 320000
F = 128
H = 64

NC = 2
NS = 16
NW = NC * NS
B = 32
NB = 324
EW = NB * B
EPAD = NW * EW
BR = 128
NBR = EW // BR
NACC = ((N + 1 + NS * 8 - 1) // (NS * 8)) * NS * 8
NPAD_ROWS = NACC - N
ROWS_PER_SUB = NACC // NS

EB = 4096


@functools.lru_cache(maxsize=None)
def _sc_mesh():
    return plsc.VectorSubcoreMesh(core_axis_name="c", subcore_axis_name="s")


def _rel_body(cx_hbm, cy_hbm, cz_hbm, src_hbm, dst_hbm, rx_hbm, ry_hbm, rz_hbm,
              srcs_v, dsts_v, s0_v, s1_v, s2_v, d0_v, d1_v, d2_v,
              o0_v, o1_v, o2_v, sem, semw):
    c = lax.axis_index("c")
    s = lax.axis_index("s")
    wid = s * NC + c
    pltpu.sync_copy(src_hbm.at[wid], srcs_v)
    pltpu.sync_copy(dst_hbm.at[wid], dsts_v)

    halves = ((s0_v.at[pl.ds(0, BR)], s1_v.at[pl.ds(0, BR)],
               s2_v.at[pl.ds(0, BR)], d0_v.at[pl.ds(0, BR)],
               d1_v.at[pl.ds(0, BR)], d2_v.at[pl.ds(0, BR)]),
              (s0_v.at[pl.ds(BR, BR)], s1_v.at[pl.ds(BR, BR)],
               s2_v.at[pl.ds(BR, BR)], d0_v.at[pl.ds(BR, BR)],
               d1_v.at[pl.ds(BR, BR)], d2_v.at[pl.ds(BR, BR)]))

    def copies(t, p):
        si = srcs_v.at[pl.ds(t * BR, BR)]
        di = dsts_v.at[pl.ds(t * BR, BR)]
        b0, b1, b2, b3, b4, b5 = halves[p]
        return (
            pltpu.make_async_copy(cx_hbm.at[si], b0, sem),
            pltpu.make_async_copy(cy_hbm.at[si], b1, sem),
            pltpu.make_async_copy(cz_hbm.at[si], b2, sem),
            pltpu.make_async_copy(cx_hbm.at[di], b3, sem),
            pltpu.make_async_copy(cy_hbm.at[di], b4, sem),
            pltpu.make_async_copy(cz_hbm.at[di], b5, sem),
        )

    out_halves = ((o0_v.at[pl.ds(0, BR)], o1_v.at[pl.ds(0, BR)],
                   o2_v.at[pl.ds(0, BR)]),
                  (o0_v.at[pl.ds(BR, BR)], o1_v.at[pl.ds(BR, BR)],
                   o2_v.at[pl.ds(BR, BR)]))

    def writes(t, p):
        base = wid * EW + t * BR
        oh = out_halves[p]
        return (
            pltpu.make_async_copy(oh[0], rx_hbm.at[pl.ds(base, BR)], semw),
            pltpu.make_async_copy(oh[1], ry_hbm.at[pl.ds(base, BR)], semw),
            pltpu.make_async_copy(oh[2], rz_hbm.at[pl.ds(base, BR)], semw),
        )

    for cp in copies(0, 0):
        cp.start()
    for cp in copies(1, 1):
        cp.start()

    def step(j, p, first):
        for cp in copies(j, p):
            cp.wait()
        if not first:
            for cp in writes(j - 2, p):
                cp.wait()
        off = p * BR
        for sbuf, dbuf, obuf in ((s0_v, d0_v, o0_v), (s1_v, d1_v, o1_v),
                                 (s2_v, d2_v, o2_v)):
            for k in range(BR // 16):
                sl = pl.ds(off + k * 16, 16)
                obuf[sl] = sbuf[sl] - dbuf[sl]
        for cp in writes(j, p):
            cp.start()

        @pl.when(j + 2 < NBR)
        def _():
            for cp in copies(j + 2, p):
                cp.start()

    step(0, 0, True)
    step(1, 1, True)

    def pair(kk, carry):
        for p in (0, 1):
            j = 2 * kk + p
            step(j, p, False)
        return carry

    lax.fori_loop(1, (NBR - 1) // 2, pair, 0)
    step(NBR - 1, (NBR - 1) % 2, False)
    for cp in writes(NBR - 2, (NBR - 2) % 2):
        cp.wait()
    for cp in writes(NBR - 1, (NBR - 1) % 2):
        cp.wait()


@functools.lru_cache(maxsize=None)
def _rel_call():
    return pl.kernel(
        _rel_body,
        out_type=tuple(jax.ShapeDtypeStruct((EPAD,), jnp.float32)
                       for _ in range(3)),
        mesh=_sc_mesh(),
        scratch_types=[
            pltpu.VMEM((EW,), jnp.int32),
            pltpu.VMEM((EW,), jnp.int32),
            pltpu.VMEM((2 * BR,), jnp.float32),
            pltpu.VMEM((2 * BR,), jnp.float32),
            pltpu.VMEM((2 * BR,), jnp.float32),
            pltpu.VMEM((2 * BR,), jnp.float32),
            pltpu.VMEM((2 * BR,), jnp.float32),
            pltpu.VMEM((2 * BR,), jnp.float32),
            pltpu.VMEM((2 * BR,), jnp.float32),
            pltpu.VMEM((2 * BR,), jnp.float32),
            pltpu.VMEM((2 * BR,), jnp.float32),
            pltpu.SemaphoreType.DMA,
            pltpu.SemaphoreType.DMA,
        ],
    )


def _msg_body(w_hbm, m_hbm, src_hbm, dst_hbm, zero_hbm, out_hbm,
              srcs_v, d0_v, d1_v, d2_v, d3_v, w0_v, w1_v, w2_v, w3_v,
              m0_v, m1_v, m2_v, m3_v, acc_sh,
              sd0, sd1, sd2, sd3, sg0, sg1, sg2, sg3,
              sw0, sw1, sw2, sw3, ss0, ss1, ss2, ss3):
    c = lax.axis_index("c")
    s = lax.axis_index("s")
    wid = s * NC + c
    pltpu.sync_copy(src_hbm.at[wid], srcs_v)
    pltpu.sync_copy(zero_hbm.at[pl.ds(s * ROWS_PER_SUB, ROWS_PER_SUB)],
                    acc_sh.at[pl.ds(s * ROWS_PER_SUB, ROWS_PER_SUB)])

    bufs = ((d0_v, w0_v, m0_v, sd0, sg0, sw0, ss0),
            (d1_v, w1_v, m1_v, sd1, sg1, sw1, ss1),
            (d2_v, w2_v, m2_v, sd2, sg2, sw2, ss2),
            (d3_v, w3_v, m3_v, sd3, sg3, sw3, ss3))

    def copies(t, p):
        d_v, w_v, m_v, sd, sg, sw, _ = bufs[p]
        return (
            pltpu.make_async_copy(dst_hbm.at[wid, pl.ds(t * B, B)], d_v, sd),
            pltpu.make_async_copy(m_hbm.at[srcs_v.at[pl.ds(t * B, B)]], m_v, sg),
            pltpu.make_async_copy(w_hbm.at[pl.ds(wid * EW + t * B, B)], w_v, sw),
        )

    def scatter_cp(p):
        d_v, _, m_v, _, _, _, ss = bufs[p]
        return pltpu.make_async_copy(m_v, acc_sh.at[d_v], ss)

    for t in (0, 1, 2):
        for cp in copies(t, t):
            cp.start()
    plsc.subcore_barrier()

    def step(j, p, first):
        for cp in copies(j, p):
            cp.wait()
        d_v, w_v, m_v = bufs[p][0], bufs[p][1], bufs[p][2]

        def row(r, c2):
            for g in range(4):
                lo = pl.ds(16 * g, 16)
                hi = pl.ds(64 + 16 * g, 16)
                bits = w_v[r, pl.ds(16 * g, 16)]
                flo = lax.bitcast_convert_type(bits << 16, jnp.float32)
                fhi = lax.bitcast_convert_type(bits & jnp.int32(-65536),
                                               jnp.float32)
                m_v[r, lo] = flo * m_v[r, lo]
                m_v[r, hi] = fhi * m_v[r, hi]
            return c2

        lax.fori_loop(0, B, row, 0)
        scatter_cp(p).start(add=True)

        @pl.when(j + 3 < NB)
        def _():
            if first:
                for cp in copies(j + 3, (p + 3) % 4):
                    cp.start()
            else:
                scatter_cp((p + 3) % 4).wait()
                for cp in copies(j + 3, (p + 3) % 4):
                    cp.start()

    step(0, 0, True)

    def quad(k, carry):
        for p in (0, 1, 2, 3):
            j = 4 * k + p

            @pl.when(jnp.logical_and(j >= 1, j < NB))
            def _():
                step(j, p, False)
        return carry

    lax.fori_loop(0, NB // 4 + 1, quad, 0)
    for t in (NB - 4, NB - 3, NB - 2, NB - 1):
        scatter_cp(t % 4).wait()
    plsc.subcore_barrier()
    pltpu.sync_copy(acc_sh.at[pl.ds(s * ROWS_PER_SUB, ROWS_PER_SUB)],
                    out_hbm.at[c, pl.ds(s * ROWS_PER_SUB, ROWS_PER_SUB)])


@functools.lru_cache(maxsize=None)
def _msg_call():
    return pl.kernel(
        _msg_body,
        out_type=jax.ShapeDtypeStruct((NC, NACC, F), jnp.float32),
        mesh=_sc_mesh(),
        scratch_types=[
            pltpu.VMEM((EW,), jnp.int32),
            pltpu.VMEM((B,), jnp.int32),
            pltpu.VMEM((B,), jnp.int32),
            pltpu.VMEM((B,), jnp.int32),
            pltpu.VMEM((B,), jnp.int32),
            pltpu.VMEM((B, F // 2), jnp.int32),
            pltpu.VMEM((B, F // 2), jnp.int32),
            pltpu.VMEM((B, F // 2), jnp.int32),
            pltpu.VMEM((B, F // 2), jnp.int32),
            pltpu.VMEM((B, F), jnp.float32),
            pltpu.VMEM((B, F), jnp.float32),
            pltpu.VMEM((B, F), jnp.float32),
            pltpu.VMEM((B, F), jnp.float32),
            pltpu.VMEM_SHARED((NACC, F), jnp.float32),
        ] + [pltpu.SemaphoreType.DMA] * 16,
    )


def _m0_body(x_ref, w_ref, b_ref, o_ref):
    o_ref[...] = jnp.dot(x_ref[...].astype(jnp.bfloat16), w_ref[...],
                         preferred_element_type=jnp.float32) + b_ref[...]


def _m1_body(p_ref, w_ref, b_ref, o_ref):
    h = (p_ref[0] + p_ref[1])[:N]
    h = jnp.where(h > 0, h, 0.01 * h)
    o_ref[...] = jnp.dot(h.astype(jnp.bfloat16), w_ref[...],
                         preferred_element_type=jnp.float32) + b_ref[...]


def _fin_body(p_ref, o_ref):
    o_ref[...] = (p_ref[0] + p_ref[1])[:N]


def _wmlp_body(rel_ref, W1_ref, b1_ref, W2_ref, b2_ref, w_ref):
    u = lax.dot_general(rel_ref[...], W1_ref[...],
                        dimension_numbers=(((0,), (0,)), ((), ())),
                        preferred_element_type=jnp.float32) + b1_ref[...]
    u = jnp.maximum(u, 0.0)
    w = jnp.dot(u.astype(jnp.bfloat16), W2_ref[...],
                preferred_element_type=jnp.float32) + b2_ref[...]
    wl = lax.bitcast_convert_type(w[:, :F // 2].astype(jnp.bfloat16),
                                  jnp.uint16).astype(jnp.int32)
    wh = lax.bitcast_convert_type(w[:, F // 2:].astype(jnp.bfloat16),
                                  jnp.uint16).astype(jnp.int32)
    w_ref[...] = wl | (wh << 16)


def _node_linear(x, Wn, bn):
    return pl.pallas_call(
        _m0_body,
        out_shape=jax.ShapeDtypeStruct((N, F), jnp.float32),
    )(x, Wn, bn.reshape(1, F))


def _node_linear_from_partials(p, Wn, bn):
    return pl.pallas_call(
        _m1_body,
        out_shape=jax.ShapeDtypeStruct((N, F), jnp.float32),
    )(p, Wn, bn.reshape(1, F))


def _combine(p):
    return pl.pallas_call(
        _fin_body,
        out_shape=jax.ShapeDtypeStruct((N, F), jnp.float32),
    )(p)


def _edge_mlp(rel, We1, be1, We2, be2):
    return pl.pallas_call(
        _wmlp_body,
        grid=(EPAD // EB,),
        in_specs=[
            pl.BlockSpec((3, EB), lambda i: (0, i)),
            pl.BlockSpec((3, H), lambda i: (0, 0)),
            pl.BlockSpec((1, H), lambda i: (0, 0)),
            pl.BlockSpec((H, F), lambda i: (0, 0)),
            pl.BlockSpec((1, F), lambda i: (0, 0)),
        ],
        out_specs=pl.BlockSpec((EB, F // 2), lambda i: (i, 0)),
        out_shape=jax.ShapeDtypeStruct((EPAD, F // 2), jnp.int32),
    )(rel, We1, be1.reshape(1, H), We2, be2.reshape(1, F))


def kernel(x, coords, edge_index, We1_0, be1_0, We2_0, be2_0, Wn_0, bn_0,
           We1_1, be1_1, We2_1, be2_1, Wn_1, bn_1):
    src = edge_index[0]
    dst = edge_index[1]
    pad = EPAD - E
    src_p = jnp.concatenate([src, jnp.arange(pad, dtype=jnp.int32) % 997])
    dummy = N + (jnp.arange(pad, dtype=jnp.int32) % NPAD_ROWS)
    dst_p = jnp.concatenate([dst, dummy])
    src2d = src_p.reshape(NW, EW)
    dst2d = dst_p.reshape(NW, EW)

    cx = coords[:, 0]
    cy = coords[:, 1]
    cz = coords[:, 2]
    zeros_acc = jnp.zeros((NACC, F), jnp.float32)

    rx, ry, rz = _rel_call()(cx, cy, cz, src2d, dst2d)
    rel = jnp.stack([rx, ry, rz])

    m0 = _node_linear(x, Wn_0.astype(jnp.bfloat16), bn_0)
    w0 = _edge_mlp(rel, We1_0, be1_0, We2_0.astype(jnp.bfloat16), be2_0)
    p0 = _msg_call()(w0, m0, src2d, dst2d, zeros_acc)

    m1 = _node_linear_from_partials(p0, Wn_1.astype(jnp.bfloat16), bn_1)
    w1 = _edge_mlp(rel, We1_1, be1_1, We2_1.astype(jnp.bfloat16), be2_1)
    p1 = _msg_call()(w1, m1, src2d, dst2d, zeros_acc)

    return _combine(p1)
